# baseline (speedup 1.0000x reference)
import jax
import jax.numpy as jnp
from jax import lax
from jax.experimental import pallas as pl
from jax.experimental.pallas import tpu as pltpu
from jax.experimental.pallas import tpu_sc as plsc

N = 10000
E = 320000
G = 64
C = 10

NC = 2
NS = 16
LANE = 128
ROWS_PER_T = 160
CH = 8
NCH = ROWS_PER_T // CH
EPAD = NS * ROWS_PER_T * LANE
NP = 10112
ROWS_PER_TILE = NP // NS


def _sc_agg_body(with_counts, src_hbm, dst_hbm, y_hbm, zeros_hbm, *refs):
    if with_counts:
        (zc_hbm, ones_hbm, agg_out, cnt_out,
         src_v, dst_v, rows0, rows1, ones_v, y_sh, acc_sh, cnt_sh,
         gs0, gs1, ss0, ss1, cs) = refs
    else:
        (agg_out,
         src_v, dst_v, rows0, rows1, y_sh, acc_sh,
         gs0, gs1, ss0, ss1, cs) = refs
    rows = (rows0, rows1)
    gsem = (gs0, gs1)
    ssem = (ss0, ss1)
    c = lax.axis_index("c")
    s = lax.axis_index("s")

    r0 = s * ROWS_PER_TILE
    pltpu.sync_copy(y_hbm.at[c, pl.ds(r0, ROWS_PER_TILE)],
                    y_sh.at[pl.ds(r0, ROWS_PER_TILE)])
    pltpu.sync_copy(zeros_hbm.at[pl.ds(r0, ROWS_PER_TILE)],
                    acc_sh.at[pl.ds(r0, ROWS_PER_TILE)])
    if with_counts:
        pltpu.sync_copy(zc_hbm.at[pl.ds(r0, ROWS_PER_TILE)],
                        cnt_sh.at[pl.ds(r0, ROWS_PER_TILE)])
        pltpu.sync_copy(ones_hbm, ones_v)
    plsc.subcore_barrier()

    def chunk(t, carry):
        pltpu.sync_copy(src_hbm.at[s, pl.ds(t * CH, CH)], src_v)
        pltpu.sync_copy(dst_hbm.at[s, pl.ds(t * CH, CH)], dst_v)
        gd = [None, None]
        sd = [None, None]
        cd = []
        gd[0] = pltpu.async_copy(y_sh.at[src_v.at[0]], rows[0], gsem[0])
        for r in range(CH):
            b = r & 1
            if r + 1 < CH:
                nb = (r + 1) & 1
                if sd[nb] is not None:
                    sd[nb].wait()
                gd[nb] = pltpu.async_copy(y_sh.at[src_v.at[r + 1]],
                                          rows[nb], gsem[nb])
            gd[b].wait()
            sd[b] = pltpu.async_copy(rows[b], acc_sh.at[dst_v.at[r]],
                                     ssem[b], add=True)
            if with_counts:
                cd.append(pltpu.async_copy(ones_v, cnt_sh.at[dst_v.at[r]],
                                           cs, add=True))
        sd[0].wait()
        sd[1].wait()
        for d in cd:
            d.wait()
        return carry

    lax.fori_loop(0, NCH, chunk, 0)
    plsc.subcore_barrier()

    pltpu.sync_copy(acc_sh.at[pl.ds(r0, ROWS_PER_TILE)],
                    agg_out.at[pl.ds(c * NP + r0, ROWS_PER_TILE)])
    if with_counts:
        pltpu.sync_copy(cnt_sh.at[pl.ds(r0, ROWS_PER_TILE)],
                        cnt_out.at[pl.ds(c * NP + r0, ROWS_PER_TILE)])


def _sc_aggregate(srcp, dstp, y_split, fh, with_counts):
    mesh = plsc.VectorSubcoreMesh(core_axis_name="c", subcore_axis_name="s",
                                  num_cores=NC, num_subcores=NS)
    zeros = jnp.zeros((NP, fh), jnp.float32)
    agg_type = jax.ShapeDtypeStruct((NC * NP, fh), jnp.float32)
    scratch = [
        pltpu.VMEM((CH, LANE), jnp.int32),
        pltpu.VMEM((CH, LANE), jnp.int32),
        pltpu.VMEM((LANE, fh), jnp.float32),
        pltpu.VMEM((LANE, fh), jnp.float32),
    ]
    args = [srcp, dstp, y_split, zeros]
    if with_counts:
        out_type = (agg_type, jax.ShapeDtypeStruct((NC * NP, 16), jnp.float32))
        scratch.append(pltpu.VMEM((LANE, 16), jnp.float32))
        args.append(jnp.zeros((NP, 16), jnp.float32))
        args.append(jnp.ones((LANE, 16), jnp.float32))
    else:
        out_type = agg_type
    scratch.append(pltpu.VMEM_SHARED((NP, fh), jnp.float32))
    scratch.append(pltpu.VMEM_SHARED((NP, fh), jnp.float32))
    if with_counts:
        scratch.append(pltpu.VMEM_SHARED((NP, 16), jnp.float32))
    for _ in range(5):
        scratch.append(pltpu.SemaphoreType.DMA)

    if with_counts:
        def body(src_hbm, dst_hbm, y_hbm, z_hbm, zc_hbm, o_hbm,
                 agg_out, cnt_out, src_v, dst_v, rows0, rows1, ones_v,
                 y_sh, acc_sh, cnt_sh, gs0, gs1, ss0, ss1, cs):
            _sc_agg_body(True, src_hbm, dst_hbm, y_hbm, z_hbm,
                         zc_hbm, o_hbm, agg_out, cnt_out,
                         src_v, dst_v, rows0, rows1, ones_v,
                         y_sh, acc_sh, cnt_sh, gs0, gs1, ss0, ss1, cs)
    else:
        def body(src_hbm, dst_hbm, y_hbm, z_hbm,
                 agg_out, src_v, dst_v, rows0, rows1, y_sh, acc_sh,
                 gs0, gs1, ss0, ss1, cs):
            _sc_agg_body(False, src_hbm, dst_hbm, y_hbm, z_hbm,
                         agg_out, src_v, dst_v, rows0, rows1, y_sh, acc_sh,
                         gs0, gs1, ss0, ss1, cs)

    fn = pl.kernel(body, out_type=out_type, mesh=mesh,
                   scratch_types=tuple(scratch),
                   compiler_params=pltpu.CompilerParams(
                       use_tc_tiling_on_sc=False))
    return fn(*args)


def _tc_call(body, out_shapes, *args):
    return pl.pallas_call(body, out_shape=out_shapes)(*args)


def _dot(a, b):
    return jax.lax.dot_general(a, b, (((1,), (0,)), ((), ())),
                               precision=jax.lax.Precision.HIGHEST,
                               preferred_element_type=jnp.float32)


def _split_for_sc(y_full, fh):
    parts = jnp.stack([y_full[:, :fh], y_full[:, fh:]])
    return jnp.concatenate(
        [parts, jnp.zeros((NC, NP - N, fh), jnp.float32)], axis=1)


def _tc_first(x, Wl1):
    def body(x_ref, w_ref, y_ref):
        y_ref[...] = _dot(x_ref[...], w_ref[...].T)
    return _tc_call(body, jax.ShapeDtypeStruct((N, 128), jnp.float32),
                    x, Wl1)


def _bn_relu(mean, h_ref, wr_ref, bl_ref, g_ref, b_ref):
    z = mean + _dot(h_ref[...], wr_ref[...].T) + bl_ref[...]
    h = jnp.maximum(z, 0.0)
    m = jnp.mean(h, axis=0, keepdims=True)
    v = jnp.mean((h - m) * (h - m), axis=0, keepdims=True)
    return g_ref[...] * (h - m) * jax.lax.rsqrt(v + 1e-5) + b_ref[...]


def _tc_layer1(agg_lo, agg_hi, cnt, x, Wr, bl, g, beta, Wl_next):
    def body(alo_ref, ahi_ref, cnt_ref, h_ref, wr_ref, bl_ref, g_ref, b_ref,
             wn_ref, h_out, y_out, inv_out):
        inv = 1.0 / jnp.maximum(cnt_ref[:N, :1], 1.0)
        agg = jnp.concatenate([alo_ref[:N, :], ahi_ref[:N, :]], axis=1)
        hn = _bn_relu(agg * inv, h_ref, wr_ref, bl_ref, g_ref, b_ref)
        h_out[...] = hn
        y_out[...] = _dot(hn, wn_ref[...].T)
        inv_out[...] = jnp.broadcast_to(inv, (N, 128))
    return _tc_call(
        body,
        (jax.ShapeDtypeStruct((N, 128), jnp.float32),
         jax.ShapeDtypeStruct((N, 64), jnp.float32),
         jax.ShapeDtypeStruct((N, 128), jnp.float32)),
        agg_lo, agg_hi, cnt, x, Wr, bl.reshape(1, -1), g.reshape(1, -1),
        beta.reshape(1, -1), Wl_next)


def _tc_layer2(agg_lo, agg_hi, inv_b, h_prev, Wr, bl, g, beta, Wl_next,
               fout, fnext):
    def body(alo_ref, ahi_ref, inv_ref, h_ref, wr_ref, bl_ref, g_ref, b_ref,
             wn_ref, h_out, y_out):
        agg = jnp.concatenate([alo_ref[:N, :], ahi_ref[:N, :]], axis=1)
        hn = _bn_relu(agg * inv_ref[:, :1], h_ref, wr_ref, bl_ref,
                      g_ref, b_ref)
        h_out[...] = hn
        y_out[...] = _dot(hn, wn_ref[...].T)
    return _tc_call(
        body,
        (jax.ShapeDtypeStruct((N, fout), jnp.float32),
         jax.ShapeDtypeStruct((N, fnext), jnp.float32)),
        agg_lo, agg_hi, inv_b, h_prev, Wr, bl.reshape(1, -1),
        g.reshape(1, -1), beta.reshape(1, -1), Wl_next)


def _tc_final(agg_lo, agg_hi, inv_b, h_prev, Wr, bl, g, beta, batch,
              fW1, fb1, fW2, fb2, fW3, fb3):
    def body(alo_ref, ahi_ref, inv_ref, h_ref, wr_ref, bl_ref, g_ref, b_ref,
             batch_ref, f1_ref, fb1_ref, f2_ref, fb2_ref, f3_ref, fb3_ref,
             out_ref):
        agg = jnp.concatenate([alo_ref[:N, :], ahi_ref[:N, :]], axis=1)
        hn = _bn_relu(agg * inv_ref[:, :1], h_ref, wr_ref, bl_ref,
                      g_ref, b_ref)
        gids = jax.lax.broadcasted_iota(jnp.int32, (G, N), 0)
        onehot = jnp.where(batch_ref[...] == gids, 1.0, 0.0)
        pooled = _dot(onehot, hn)
        cr = jnp.maximum(_dot(pooled, f1_ref[...].T) + fb1_ref[...], 0.0)
        cr = jnp.maximum(_dot(cr, f2_ref[...].T) + fb2_ref[...], 0.0)
        out_ref[...] = _dot(cr, f3_ref[...].T) + fb3_ref[...]
    return _tc_call(
        body, jax.ShapeDtypeStruct((G, C), jnp.float32),
        agg_lo, agg_hi, inv_b, h_prev, Wr, bl.reshape(1, -1),
        g.reshape(1, -1), beta.reshape(1, -1), batch.reshape(1, N),
        fW1, fb1.reshape(1, -1), fW2, fb2.reshape(1, -1),
        fW3, fb3.reshape(1, -1))


def kernel(x, edge_index, batch, Wl1, bl1, Wr1, Wl2, bl2, Wr2, Wl3, bl3, Wr3,
           g1, beta1, g2, beta2, g3, beta3, fW1, fb1, fW2, fb2, fW3, fb3):
    edge_index = edge_index.astype(jnp.int32)
    batch = batch.astype(jnp.int32)
    src = jnp.concatenate(
        [edge_index[0], jnp.zeros((EPAD - E,), jnp.int32)]).reshape(
            NS, ROWS_PER_T, LANE)
    dst = jnp.concatenate(
        [edge_index[1], jnp.full((EPAD - E,), N, jnp.int32)]).reshape(
            NS, ROWS_PER_T, LANE)

    y1 = _tc_first(x, Wl1)
    agg1, cnt = _sc_aggregate(src, dst, _split_for_sc(y1, 64), 64, True)
    h1, y2, inv_b = _tc_layer1(agg1[:NP], agg1[NP:], cnt[:NP], x,
                               Wr1, bl1, g1, beta1, Wl2)
    agg2 = _sc_aggregate(src, dst, _split_for_sc(y2, 32), 32, False)
    h2, y3 = _tc_layer2(agg2[:NP], agg2[NP:], inv_b, h1,
                        Wr2, bl2, g2, beta2, Wl3, 64, 32)
    agg3 = _sc_aggregate(src, dst, _split_for_sc(y3, 16), 16, False)
    return _tc_final(agg3[:NP], agg3[NP:], inv_b, h2, Wr3, bl3, g3, beta3,
                     batch, fW1, fb1, fW2, fb2, fW3, fb3)


# speedup vs baseline: 1.1078x; 1.1078x over previous
import jax
import jax.numpy as jnp
from jax import lax
from jax.experimental import pallas as pl
from jax.experimental.pallas import tpu as pltpu
from jax.experimental.pallas import tpu_sc as plsc

N = 10000
E = 320000
G = 64
C = 10

NC = 2
NS = 16
LANE = 64
ROWS_PER_T = 320
CH = 32
NB = 4
NCH = ROWS_PER_T // CH
EPAD = NS * ROWS_PER_T * LANE
NP = 10112
ROWS_PER_TILE = NP // NS


def _sc_agg_body(with_counts, src_hbm, dst_hbm, y_hbm, zeros_hbm, *refs):
    if with_counts:
        (zc_hbm, ones_hbm, agg_out, cnt_out, src_v, dst_v) = refs[:6]
        rows = refs[6:6 + NB]
        (ones_v, y_sh, acc_sh, cnt_sh) = refs[6 + NB:10 + NB]
        gsem = refs[10 + NB:10 + 2 * NB]
        ssem = refs[10 + 2 * NB:10 + 3 * NB]
        cs = refs[10 + 3 * NB]
    else:
        (agg_out, src_v, dst_v) = refs[:3]
        rows = refs[3:3 + NB]
        (y_sh, acc_sh) = refs[3 + NB:5 + NB]
        gsem = refs[5 + NB:5 + 2 * NB]
        ssem = refs[5 + 2 * NB:5 + 3 * NB]
        cs = refs[5 + 3 * NB]
    c = lax.axis_index("c")
    s = lax.axis_index("s")

    r0 = s * ROWS_PER_TILE
    pltpu.sync_copy(y_hbm.at[c, pl.ds(r0, ROWS_PER_TILE)],
                    y_sh.at[pl.ds(r0, ROWS_PER_TILE)])
    pltpu.sync_copy(zeros_hbm.at[pl.ds(r0, ROWS_PER_TILE)],
                    acc_sh.at[pl.ds(r0, ROWS_PER_TILE)])
    if with_counts:
        pltpu.sync_copy(zc_hbm.at[pl.ds(r0, ROWS_PER_TILE)],
                        cnt_sh.at[pl.ds(r0, ROWS_PER_TILE)])
        pltpu.sync_copy(ones_hbm, ones_v)
    plsc.subcore_barrier()

    def chunk(t, carry):
        pltpu.sync_copy(src_hbm.at[s, pl.ds(t * CH, CH)], src_v)
        pltpu.sync_copy(dst_hbm.at[s, pl.ds(t * CH, CH)], dst_v)
        gd = [None] * NB
        sd = [None] * NB
        cd = []
        for k in range(NB - 1):
            gd[k] = pltpu.async_copy(y_sh.at[src_v.at[k]], rows[k], gsem[k])
        for r in range(CH):
            b = r % NB
            ahead = r + NB - 1
            if ahead < CH:
                nb = ahead % NB
                if sd[nb] is not None:
                    sd[nb].wait()
                    sd[nb] = None
                gd[nb] = pltpu.async_copy(y_sh.at[src_v.at[ahead]],
                                          rows[nb], gsem[nb])
            gd[b].wait()
            sd[b] = pltpu.async_copy(rows[b], acc_sh.at[dst_v.at[r]],
                                     ssem[b], add=True)
            if with_counts:
                cd.append(pltpu.async_copy(ones_v, cnt_sh.at[dst_v.at[r]],
                                           cs, add=True))
        for d in sd:
            if d is not None:
                d.wait()
        for d in cd:
            d.wait()
        return carry

    lax.fori_loop(0, NCH, chunk, 0)
    plsc.subcore_barrier()

    pltpu.sync_copy(acc_sh.at[pl.ds(r0, ROWS_PER_TILE)],
                    agg_out.at[pl.ds(c * NP + r0, ROWS_PER_TILE)])
    if with_counts:
        pltpu.sync_copy(cnt_sh.at[pl.ds(r0, ROWS_PER_TILE)],
                        cnt_out.at[pl.ds(c * NP + r0, ROWS_PER_TILE)])


def _sc_aggregate(srcp, dstp, y_split, fh, with_counts):
    mesh = plsc.VectorSubcoreMesh(core_axis_name="c", subcore_axis_name="s",
                                  num_cores=NC, num_subcores=NS)
    zeros = jnp.zeros((NP, fh), jnp.float32)
    agg_type = jax.ShapeDtypeStruct((NC * NP, fh), jnp.float32)
    scratch = [
        pltpu.VMEM((CH, LANE), jnp.int32),
        pltpu.VMEM((CH, LANE), jnp.int32),
    ]
    for _ in range(NB):
        scratch.append(pltpu.VMEM((LANE, fh), jnp.float32))
    args = [srcp, dstp, y_split, zeros]
    if with_counts:
        out_type = (agg_type, jax.ShapeDtypeStruct((NC * NP, 16), jnp.float32))
        scratch.append(pltpu.VMEM((LANE, 16), jnp.float32))
        args.append(jnp.zeros((NP, 16), jnp.float32))
        args.append(jnp.ones((LANE, 16), jnp.float32))
    else:
        out_type = agg_type
    scratch.append(pltpu.VMEM_SHARED((NP, fh), jnp.float32))
    scratch.append(pltpu.VMEM_SHARED((NP, fh), jnp.float32))
    if with_counts:
        scratch.append(pltpu.VMEM_SHARED((NP, 16), jnp.float32))
    for _ in range(2 * NB + 1):
        scratch.append(pltpu.SemaphoreType.DMA)

    nin = 6 if with_counts else 4

    def body(*allrefs):
        _sc_agg_body(with_counts, allrefs[0], allrefs[1], allrefs[2],
                     allrefs[3], *allrefs[4:nin], *allrefs[nin:])

    fn = pl.kernel(body, out_type=out_type, mesh=mesh,
                   scratch_types=tuple(scratch),
                   compiler_params=pltpu.CompilerParams(
                       use_tc_tiling_on_sc=False))
    return fn(*args)


def _tc_call(body, out_shapes, *args):
    return pl.pallas_call(body, out_shape=out_shapes)(*args)


def _dot(a, b):
    return jax.lax.dot_general(a, b, (((1,), (0,)), ((), ())),
                               precision=jax.lax.Precision.HIGHEST,
                               preferred_element_type=jnp.float32)


def _split_for_sc(y_full, fh):
    parts = jnp.stack([y_full[:, :fh], y_full[:, fh:]])
    return jnp.concatenate(
        [parts, jnp.zeros((NC, NP - N, fh), jnp.float32)], axis=1)


def _tc_first(x, Wl1):
    def body(x_ref, w_ref, y_ref):
        y_ref[...] = _dot(x_ref[...], w_ref[...].T)
    return _tc_call(body, jax.ShapeDtypeStruct((N, 128), jnp.float32),
                    x, Wl1)


def _bn_relu(mean, h_ref, wr_ref, bl_ref, g_ref, b_ref):
    z = mean + _dot(h_ref[...], wr_ref[...].T) + bl_ref[...]
    h = jnp.maximum(z, 0.0)
    m = jnp.mean(h, axis=0, keepdims=True)
    v = jnp.mean((h - m) * (h - m), axis=0, keepdims=True)
    return g_ref[...] * (h - m) * jax.lax.rsqrt(v + 1e-5) + b_ref[...]


def _tc_layer1(agg_lo, agg_hi, cnt, x, Wr, bl, g, beta, Wl_next):
    def body(alo_ref, ahi_ref, cnt_ref, h_ref, wr_ref, bl_ref, g_ref, b_ref,
             wn_ref, h_out, y_out, inv_out):
        inv = 1.0 / jnp.maximum(cnt_ref[:N, :1], 1.0)
        agg = jnp.concatenate([alo_ref[:N, :], ahi_ref[:N, :]], axis=1)
        hn = _bn_relu(agg * inv, h_ref, wr_ref, bl_ref, g_ref, b_ref)
        h_out[...] = hn
        y_out[...] = _dot(hn, wn_ref[...].T)
        inv_out[...] = jnp.broadcast_to(inv, (N, 128))
    return _tc_call(
        body,
        (jax.ShapeDtypeStruct((N, 128), jnp.float32),
         jax.ShapeDtypeStruct((N, 64), jnp.float32),
         jax.ShapeDtypeStruct((N, 128), jnp.float32)),
        agg_lo, agg_hi, cnt, x, Wr, bl.reshape(1, -1), g.reshape(1, -1),
        beta.reshape(1, -1), Wl_next)


def _tc_layer2(agg_lo, agg_hi, inv_b, h_prev, Wr, bl, g, beta, Wl_next,
               fout, fnext):
    def body(alo_ref, ahi_ref, inv_ref, h_ref, wr_ref, bl_ref, g_ref, b_ref,
             wn_ref, h_out, y_out):
        agg = jnp.concatenate([alo_ref[:N, :], ahi_ref[:N, :]], axis=1)
        hn = _bn_relu(agg * inv_ref[:, :1], h_ref, wr_ref, bl_ref,
                      g_ref, b_ref)
        h_out[...] = hn
        y_out[...] = _dot(hn, wn_ref[...].T)
    return _tc_call(
        body,
        (jax.ShapeDtypeStruct((N, fout), jnp.float32),
         jax.ShapeDtypeStruct((N, fnext), jnp.float32)),
        agg_lo, agg_hi, inv_b, h_prev, Wr, bl.reshape(1, -1),
        g.reshape(1, -1), beta.reshape(1, -1), Wl_next)


def _tc_final(agg_lo, agg_hi, inv_b, h_prev, Wr, bl, g, beta, batch,
              fW1, fb1, fW2, fb2, fW3, fb3):
    def body(alo_ref, ahi_ref, inv_ref, h_ref, wr_ref, bl_ref, g_ref, b_ref,
             batch_ref, f1_ref, fb1_ref, f2_ref, fb2_ref, f3_ref, fb3_ref,
             out_ref):
        agg = jnp.concatenate([alo_ref[:N, :], ahi_ref[:N, :]], axis=1)
        hn = _bn_relu(agg * inv_ref[:, :1], h_ref, wr_ref, bl_ref,
                      g_ref, b_ref)
        gids = jax.lax.broadcasted_iota(jnp.int32, (G, N), 0)
        onehot = jnp.where(batch_ref[...] == gids, 1.0, 0.0)
        pooled = _dot(onehot, hn)
        cr = jnp.maximum(_dot(pooled, f1_ref[...].T) + fb1_ref[...], 0.0)
        cr = jnp.maximum(_dot(cr, f2_ref[...].T) + fb2_ref[...], 0.0)
        out_ref[...] = _dot(cr, f3_ref[...].T) + fb3_ref[...]
    return _tc_call(
        body, jax.ShapeDtypeStruct((G, C), jnp.float32),
        agg_lo, agg_hi, inv_b, h_prev, Wr, bl.reshape(1, -1),
        g.reshape(1, -1), beta.reshape(1, -1), batch.reshape(1, N),
        fW1, fb1.reshape(1, -1), fW2, fb2.reshape(1, -1),
        fW3, fb3.reshape(1, -1))


def kernel(x, edge_index, batch, Wl1, bl1, Wr1, Wl2, bl2, Wr2, Wl3, bl3, Wr3,
           g1, beta1, g2, beta2, g3, beta3, fW1, fb1, fW2, fb2, fW3, fb3):
    edge_index = edge_index.astype(jnp.int32)
    batch = batch.astype(jnp.int32)
    src = jnp.concatenate(
        [edge_index[0], jnp.zeros((EPAD - E,), jnp.int32)]).reshape(
            NS, ROWS_PER_T, LANE)
    dst = jnp.concatenate(
        [edge_index[1], jnp.full((EPAD - E,), N, jnp.int32)]).reshape(
            NS, ROWS_PER_T, LANE)

    y1 = _tc_first(x, Wl1)
    agg1, cnt = _sc_aggregate(src, dst, _split_for_sc(y1, 64), 64, True)
    h1, y2, inv_b = _tc_layer1(agg1[:NP], agg1[NP:], cnt[:NP], x,
                               Wr1, bl1, g1, beta1, Wl2)
    agg2 = _sc_aggregate(src, dst, _split_for_sc(y2, 32), 32, False)
    h2, y3 = _tc_layer2(agg2[:NP], agg2[NP:], inv_b, h1,
                        Wr2, bl2, g2, beta2, Wl3, 64, 32)
    agg3 = _sc_aggregate(src, dst, _split_for_sc(y3, 16), 16, False)
    return _tc_final(agg3[:NP], agg3[NP:], inv_b, h2, Wr3, bl3, g3, beta3,
                     batch, fW1, fb1, fW2, fb2, fW3, fb3)
